# initial kernel scaffold (unmeasured)
import jax
import jax.numpy as jnp
from jax import lax
from jax.experimental import pallas as pl
from jax.experimental.pallas import tpu as pltpu

N_DEV = 16


def kernel(A, B):
    m_per, k = A.shape
    _, n = B.shape
    out_m = N_DEV * m_per

    def body(a_ref, b_ref, out_ref, comm_ref, send_sems, recv_sems,
             out_cp_sem, credit_sem):
        my = lax.axis_index("i")
        left = (my - 1) % N_DEV
        right = (my + 1) % N_DEV

        barrier_sem = pltpu.get_barrier_semaphore()
        for nbr in (left, right):
            pl.semaphore_signal(
                barrier_sem, inc=1,
                device_id=(nbr,), device_id_type=pl.DeviceIdType.MESH,
            )
        pl.semaphore_wait(barrier_sem, 2)

        comm_ref[0, :, :] = jnp.dot(
            a_ref[:, :], b_ref[:, :], preferred_element_type=jnp.float32
        )

        cp = pltpu.make_async_copy(
            comm_ref.at[0], out_ref.at[pl.ds(my * m_per, m_per), :], out_cp_sem
        )
        cp.start()
        cp.wait()

        pl.semaphore_signal(
            credit_sem, inc=1,
            device_id=(left,), device_id_type=pl.DeviceIdType.MESH,
        )

        for h in range(N_DEV - 1):
            send_slot = h % 2
            recv_slot = (h + 1) % 2

            pl.semaphore_wait(credit_sem, 1)

            rdma = pltpu.make_async_remote_copy(
                src_ref=comm_ref.at[send_slot],
                dst_ref=comm_ref.at[recv_slot],
                send_sem=send_sems.at[send_slot],
                recv_sem=recv_sems.at[recv_slot],
                device_id=(right,),
                device_id_type=pl.DeviceIdType.MESH,
            )
            rdma.start()
            rdma.wait()

            if h < N_DEV - 2:
                pl.semaphore_signal(
                    credit_sem, inc=1,
                    device_id=(left,), device_id_type=pl.DeviceIdType.MESH,
                )

            origin = (my - h - 1) % N_DEV
            cp = pltpu.make_async_copy(
                comm_ref.at[recv_slot],
                out_ref.at[pl.ds(origin * m_per, m_per), :],
                out_cp_sem,
            )
            cp.start()
            cp.wait()

    return pl.pallas_call(
        body,
        out_shape=jax.ShapeDtypeStruct((out_m, n), jnp.float32),
        in_specs=[
            pl.BlockSpec(memory_space=pltpu.VMEM),
            pl.BlockSpec(memory_space=pltpu.VMEM),
        ],
        out_specs=pl.BlockSpec(memory_space=pltpu.ANY),
        scratch_shapes=[
            pltpu.VMEM((2, m_per, n), jnp.float32),
            pltpu.SemaphoreType.DMA((2,)),
            pltpu.SemaphoreType.DMA((2,)),
            pltpu.SemaphoreType.DMA,
            pltpu.SemaphoreType.REGULAR,
        ],
        compiler_params=pltpu.CompilerParams(collective_id=0),
    )(A, B)


# baseline (device time: 810097 ns/iter reference)
import jax
import jax.numpy as jnp
from jax import lax
from jax.experimental import pallas as pl
from jax.experimental.pallas import tpu as pltpu

N_DEV = 16


def kernel(A, B):
    m_per, k = A.shape
    _, n = B.shape
    out_m = N_DEV * m_per

    def body(a_ref, b_ref, out_ref, comm_ref, send_sems, recv_sems,
             out_cp_sem, credit_sem):
        my = lax.axis_index("i")
        left = (my - 1) % N_DEV
        right = (my + 1) % N_DEV

        barrier_sem = pltpu.get_barrier_semaphore()
        for nbr in (left, right):
            pl.semaphore_signal(
                barrier_sem, inc=1,
                device_id=(nbr,), device_id_type=pl.DeviceIdType.MESH,
            )
        pl.semaphore_wait(barrier_sem, 2)

        comm_ref[0, :, :] = jnp.dot(
            a_ref[:, :], b_ref[:, :], preferred_element_type=jnp.float32
        )

        cp = pltpu.make_async_copy(
            comm_ref.at[0], out_ref.at[pl.ds(my * m_per, m_per), :], out_cp_sem
        )
        cp.start()
        cp.wait()

        pl.semaphore_signal(
            credit_sem, inc=1,
            device_id=(left,), device_id_type=pl.DeviceIdType.MESH,
        )

        for h in range(N_DEV - 1):
            send_slot = h % 2
            recv_slot = (h + 1) % 2

            pl.semaphore_wait(credit_sem, 1)

            rdma = pltpu.make_async_remote_copy(
                src_ref=comm_ref.at[send_slot],
                dst_ref=comm_ref.at[recv_slot],
                send_sem=send_sems.at[send_slot],
                recv_sem=recv_sems.at[recv_slot],
                device_id=(right,),
                device_id_type=pl.DeviceIdType.MESH,
            )
            rdma.start()
            rdma.wait()

            if h < N_DEV - 2:
                pl.semaphore_signal(
                    credit_sem, inc=1,
                    device_id=(left,), device_id_type=pl.DeviceIdType.MESH,
                )

            origin = (my - h - 1) % N_DEV
            cp = pltpu.make_async_copy(
                comm_ref.at[recv_slot],
                out_ref.at[pl.ds(origin * m_per, m_per), :],
                out_cp_sem,
            )
            cp.start()
            cp.wait()

    return pl.pallas_call(
        body,
        out_shape=jax.ShapeDtypeStruct((out_m, n), jnp.float32),
        in_specs=[
            pl.BlockSpec(memory_space=pltpu.VMEM),
            pl.BlockSpec(memory_space=pltpu.VMEM),
        ],
        out_specs=pl.BlockSpec(memory_space=pl.ANY),
        scratch_shapes=[
            pltpu.VMEM((2, m_per, n), jnp.float32),
            pltpu.SemaphoreType.DMA((2,)),
            pltpu.SemaphoreType.DMA((2,)),
            pltpu.SemaphoreType.DMA,
            pltpu.SemaphoreType.REGULAR,
        ],
        compiler_params=pltpu.CompilerParams(collective_id=0),
    )(A, B)


# device time: 457959 ns/iter; 1.7689x vs baseline; 1.7689x over previous
import jax
import jax.numpy as jnp
from jax import lax
from jax.experimental import pallas as pl
from jax.experimental.pallas import tpu as pltpu

N_DEV = 16
CW_HOPS = 8
CCW_HOPS = 7


def kernel(A, B):
    m_per, k = A.shape
    _, n = B.shape
    out_m = N_DEV * m_per

    def body(a_ref, b_ref, out_ref,
             cw_ref, ccw_ref,
             cw_send, cw_recv, ccw_send, ccw_recv,
             own_cp_sem, cw_cp_sems, ccw_cp_sems,
             cw_credit, ccw_credit):
        my = lax.axis_index("i")
        left = (my - 1) % N_DEV
        right = (my + 1) % N_DEV

        barrier_sem = pltpu.get_barrier_semaphore()
        for nbr in (left, right):
            pl.semaphore_signal(
                barrier_sem, inc=1,
                device_id=(nbr,), device_id_type=pl.DeviceIdType.MESH,
            )
        pl.semaphore_wait(barrier_sem, 2)

        c = jnp.dot(a_ref[:, :], b_ref[:, :],
                    preferred_element_type=jnp.float32)
        cw_ref[0, :, :] = c
        ccw_ref[0, :, :] = c

        own_cp = pltpu.make_async_copy(
            cw_ref.at[0], out_ref.at[pl.ds(my * m_per, m_per), :], own_cp_sem
        )
        own_cp.start()

        pl.semaphore_signal(
            cw_credit, inc=1,
            device_id=(left,), device_id_type=pl.DeviceIdType.MESH,
        )
        pl.semaphore_signal(
            ccw_credit, inc=1,
            device_id=(right,), device_id_type=pl.DeviceIdType.MESH,
        )

        cw_pending = own_cp
        ccw_pending = None

        for h in range(CW_HOPS):
            ss = h % 2
            rs = (h + 1) % 2

            pl.semaphore_wait(cw_credit, 1)
            cw_rdma = pltpu.make_async_remote_copy(
                src_ref=cw_ref.at[ss],
                dst_ref=cw_ref.at[rs],
                send_sem=cw_send.at[ss],
                recv_sem=cw_recv.at[rs],
                device_id=(right,),
                device_id_type=pl.DeviceIdType.MESH,
            )
            cw_rdma.start()
            if h < CCW_HOPS:
                pl.semaphore_wait(ccw_credit, 1)
                ccw_rdma = pltpu.make_async_remote_copy(
                    src_ref=ccw_ref.at[ss],
                    dst_ref=ccw_ref.at[rs],
                    send_sem=ccw_send.at[ss],
                    recv_sem=ccw_recv.at[rs],
                    device_id=(left,),
                    device_id_type=pl.DeviceIdType.MESH,
                )
                ccw_rdma.start()

            cw_rdma.wait()
            if cw_pending is not None:
                cw_pending.wait()
            if h < CW_HOPS - 1:
                pl.semaphore_signal(
                    cw_credit, inc=1,
                    device_id=(left,), device_id_type=pl.DeviceIdType.MESH,
                )
            origin = (my - h - 1) % N_DEV
            cw_cp = pltpu.make_async_copy(
                cw_ref.at[rs],
                out_ref.at[pl.ds(origin * m_per, m_per), :],
                cw_cp_sems.at[rs],
            )
            cw_cp.start()
            cw_pending = cw_cp

            if h < CCW_HOPS:
                ccw_rdma.wait()
                if ccw_pending is not None:
                    ccw_pending.wait()
                if h < CCW_HOPS - 1:
                    pl.semaphore_signal(
                        ccw_credit, inc=1,
                        device_id=(right,), device_id_type=pl.DeviceIdType.MESH,
                    )
                origin = (my + h + 1) % N_DEV
                ccw_cp = pltpu.make_async_copy(
                    ccw_ref.at[rs],
                    out_ref.at[pl.ds(origin * m_per, m_per), :],
                    ccw_cp_sems.at[rs],
                )
                ccw_cp.start()
                ccw_pending = ccw_cp

        cw_pending.wait()
        ccw_pending.wait()

    return pl.pallas_call(
        body,
        out_shape=jax.ShapeDtypeStruct((out_m, n), jnp.float32),
        in_specs=[
            pl.BlockSpec(memory_space=pltpu.VMEM),
            pl.BlockSpec(memory_space=pltpu.VMEM),
        ],
        out_specs=pl.BlockSpec(memory_space=pl.ANY),
        scratch_shapes=[
            pltpu.VMEM((2, m_per, n), jnp.float32),
            pltpu.VMEM((2, m_per, n), jnp.float32),
            pltpu.SemaphoreType.DMA((2,)),
            pltpu.SemaphoreType.DMA((2,)),
            pltpu.SemaphoreType.DMA((2,)),
            pltpu.SemaphoreType.DMA((2,)),
            pltpu.SemaphoreType.DMA,
            pltpu.SemaphoreType.DMA((2,)),
            pltpu.SemaphoreType.DMA((2,)),
            pltpu.SemaphoreType.REGULAR,
            pltpu.SemaphoreType.REGULAR,
        ],
        compiler_params=pltpu.CompilerParams(collective_id=0),
    )(A, B)


# device time: 278466 ns/iter; 2.9091x vs baseline; 1.6446x over previous
import jax
import jax.numpy as jnp
from jax import lax
from jax.experimental import pallas as pl
from jax.experimental.pallas import tpu as pltpu

N_DEV = 16
CW_HOPS = 8
CCW_HOPS = 7


def kernel(A, B):
    m_per, k = A.shape
    _, n = B.shape
    out_m = N_DEV * m_per

    def body(a_ref, b_ref, out_ref,
             cw_ref, ccw_ref,
             cw_send, cw_recv, ccw_send, ccw_recv,
             res_cw, res_ccw, cw_st_sems, ccw_st_sems,
             cw_credit, ccw_credit):
        my = lax.axis_index("i")
        left = (my - 1) % N_DEV
        right = (my + 1) % N_DEV

        barrier_sem = pltpu.get_barrier_semaphore()
        for nbr in (left, right):
            pl.semaphore_signal(
                barrier_sem, inc=1,
                device_id=(nbr,), device_id_type=pl.DeviceIdType.MESH,
            )
        pl.semaphore_wait(barrier_sem, 2)

        cw_ref[0, :, :] = a_ref[:, :]
        ccw_ref[0, :, :] = a_ref[:, :]

        pl.semaphore_signal(
            cw_credit, inc=1,
            device_id=(left,), device_id_type=pl.DeviceIdType.MESH,
        )
        pl.semaphore_signal(
            ccw_credit, inc=1,
            device_id=(right,), device_id_type=pl.DeviceIdType.MESH,
        )

        cw_store = [None, None]
        ccw_store = [None, None]

        def drain(store, slot):
            if store[slot] is not None:
                store[slot].wait()
                store[slot] = None

        def compute_and_store(src_ref, res, store, st_sems, slot, origin):
            drain(store, slot)
            res[slot, :, :] = jnp.dot(
                src_ref[:, :], b_ref[:, :],
                preferred_element_type=jnp.float32,
            )
            cp = pltpu.make_async_copy(
                res.at[slot],
                out_ref.at[pl.ds(origin * m_per, m_per), :],
                st_sems.at[slot],
            )
            cp.start()
            store[slot] = cp

        pending = [("own", None)]

        for h in range(CW_HOPS):
            ss = h % 2
            rs = (h + 1) % 2
            res_slot = h % 2

            pl.semaphore_wait(cw_credit, 1)
            cw_rdma = pltpu.make_async_remote_copy(
                src_ref=cw_ref.at[ss],
                dst_ref=cw_ref.at[rs],
                send_sem=cw_send.at[ss],
                recv_sem=cw_recv.at[rs],
                device_id=(right,),
                device_id_type=pl.DeviceIdType.MESH,
            )
            cw_rdma.start()
            if h < CCW_HOPS:
                pl.semaphore_wait(ccw_credit, 1)
                ccw_rdma = pltpu.make_async_remote_copy(
                    src_ref=ccw_ref.at[ss],
                    dst_ref=ccw_ref.at[rs],
                    send_sem=ccw_send.at[ss],
                    recv_sem=ccw_recv.at[rs],
                    device_id=(left,),
                    device_id_type=pl.DeviceIdType.MESH,
                )
                ccw_rdma.start()

            for kind, origin in pending:
                if kind == "own":
                    compute_and_store(a_ref, res_cw, cw_store,
                                      cw_st_sems, res_slot, my)
                elif kind == "cw":
                    compute_and_store(cw_ref.at[ss], res_cw, cw_store,
                                      cw_st_sems, res_slot, origin)
                else:
                    compute_and_store(ccw_ref.at[ss], res_ccw, ccw_store,
                                      ccw_st_sems, res_slot, origin)
            pending = []

            cw_rdma.wait()
            if h < CW_HOPS - 1:
                pl.semaphore_signal(
                    cw_credit, inc=1,
                    device_id=(left,), device_id_type=pl.DeviceIdType.MESH,
                )
            pending.append(("cw", (my - h - 1) % N_DEV))

            if h < CCW_HOPS:
                ccw_rdma.wait()
                if h < CCW_HOPS - 1:
                    pl.semaphore_signal(
                        ccw_credit, inc=1,
                        device_id=(right,), device_id_type=pl.DeviceIdType.MESH,
                    )
                pending.append(("ccw", (my + h + 1) % N_DEV))

        res_slot = CW_HOPS % 2
        for kind, origin in pending:
            if kind == "cw":
                compute_and_store(cw_ref.at[CW_HOPS % 2], res_cw, cw_store,
                                  cw_st_sems, res_slot, origin)
            else:
                compute_and_store(ccw_ref.at[CW_HOPS % 2], res_ccw, ccw_store,
                                  ccw_st_sems, res_slot, origin)
        for slot in (0, 1):
            drain(cw_store, slot)
            drain(ccw_store, slot)

    return pl.pallas_call(
        body,
        out_shape=jax.ShapeDtypeStruct((out_m, n), jnp.float32),
        in_specs=[
            pl.BlockSpec(memory_space=pltpu.VMEM),
            pl.BlockSpec(memory_space=pltpu.VMEM),
        ],
        out_specs=pl.BlockSpec(memory_space=pl.ANY),
        scratch_shapes=[
            pltpu.VMEM((2, m_per, k), jnp.float32),
            pltpu.VMEM((2, m_per, k), jnp.float32),
            pltpu.SemaphoreType.DMA((2,)),
            pltpu.SemaphoreType.DMA((2,)),
            pltpu.SemaphoreType.DMA((2,)),
            pltpu.SemaphoreType.DMA((2,)),
            pltpu.VMEM((2, m_per, n), jnp.float32),
            pltpu.VMEM((2, m_per, n), jnp.float32),
            pltpu.SemaphoreType.DMA((2,)),
            pltpu.SemaphoreType.DMA((2,)),
            pltpu.SemaphoreType.REGULAR,
            pltpu.SemaphoreType.REGULAR,
        ],
        compiler_params=pltpu.CompilerParams(collective_id=0),
    )(A, B)


# device time: 188412 ns/iter; 4.2996x vs baseline; 1.4780x over previous
import jax
import jax.numpy as jnp
from jax import lax
from jax.experimental import pallas as pl
from jax.experimental.pallas import tpu as pltpu

N_DEV = 16
CW_HOPS = 8
CCW_HOPS = 7


def kernel(A, B):
    m_per, k = A.shape
    _, n = B.shape
    out_m = N_DEV * m_per

    def body(a_ref, b_ref, out_ref,
             cw_ref, ccw_ref,
             cw_send, cw_recv, ccw_send, ccw_recv,
             res_cw, res_ccw, cw_st_sems, ccw_st_sems,
             b_bf, cw_credit, ccw_credit):
        my = lax.axis_index("i")
        left = (my - 1) % N_DEV
        right = (my + 1) % N_DEV

        barrier_sem = pltpu.get_barrier_semaphore()
        for nbr in (left, right):
            pl.semaphore_signal(
                barrier_sem, inc=1,
                device_id=(nbr,), device_id_type=pl.DeviceIdType.MESH,
            )
        pl.semaphore_wait(barrier_sem, 2)

        a_bf = a_ref[:, :].astype(jnp.bfloat16)
        b_bf[:, :] = b_ref[:, :].astype(jnp.bfloat16)

        cw_ref[0, :, :] = a_bf
        ccw_ref[0, :, :] = a_bf

        pl.semaphore_signal(
            cw_credit, inc=1,
            device_id=(left,), device_id_type=pl.DeviceIdType.MESH,
        )
        pl.semaphore_signal(
            ccw_credit, inc=1,
            device_id=(right,), device_id_type=pl.DeviceIdType.MESH,
        )

        cw_store = [None, None]
        ccw_store = [None, None]

        def drain(store, slot):
            if store[slot] is not None:
                store[slot].wait()
                store[slot] = None

        def compute_and_store(src_ref, res, store, st_sems, slot, origin):
            drain(store, slot)
            res[slot, :, :] = jnp.dot(
                src_ref[:, :], b_bf[:, :],
                preferred_element_type=jnp.float32,
            )
            cp = pltpu.make_async_copy(
                res.at[slot],
                out_ref.at[pl.ds(origin * m_per, m_per), :],
                st_sems.at[slot],
            )
            cp.start()
            store[slot] = cp

        pending = [("own", None)]

        for h in range(CW_HOPS):
            ss = h % 2
            rs = (h + 1) % 2
            res_slot = h % 2

            pl.semaphore_wait(cw_credit, 1)
            cw_rdma = pltpu.make_async_remote_copy(
                src_ref=cw_ref.at[ss],
                dst_ref=cw_ref.at[rs],
                send_sem=cw_send.at[ss],
                recv_sem=cw_recv.at[rs],
                device_id=(right,),
                device_id_type=pl.DeviceIdType.MESH,
            )
            cw_rdma.start()
            if h < CCW_HOPS:
                pl.semaphore_wait(ccw_credit, 1)
                ccw_rdma = pltpu.make_async_remote_copy(
                    src_ref=ccw_ref.at[ss],
                    dst_ref=ccw_ref.at[rs],
                    send_sem=ccw_send.at[ss],
                    recv_sem=ccw_recv.at[rs],
                    device_id=(left,),
                    device_id_type=pl.DeviceIdType.MESH,
                )
                ccw_rdma.start()

            for kind, origin in pending:
                if kind == "own":
                    compute_and_store(cw_ref.at[0], res_cw, cw_store,
                                      cw_st_sems, res_slot, my)
                elif kind == "cw":
                    compute_and_store(cw_ref.at[ss], res_cw, cw_store,
                                      cw_st_sems, res_slot, origin)
                else:
                    compute_and_store(ccw_ref.at[ss], res_ccw, ccw_store,
                                      ccw_st_sems, res_slot, origin)
            pending = []

            cw_rdma.wait()
            if h < CW_HOPS - 1:
                pl.semaphore_signal(
                    cw_credit, inc=1,
                    device_id=(left,), device_id_type=pl.DeviceIdType.MESH,
                )
            pending.append(("cw", (my - h - 1) % N_DEV))

            if h < CCW_HOPS:
                ccw_rdma.wait()
                if h < CCW_HOPS - 1:
                    pl.semaphore_signal(
                        ccw_credit, inc=1,
                        device_id=(right,), device_id_type=pl.DeviceIdType.MESH,
                    )
                pending.append(("ccw", (my + h + 1) % N_DEV))

        res_slot = CW_HOPS % 2
        for kind, origin in pending:
            if kind == "cw":
                compute_and_store(cw_ref.at[CW_HOPS % 2], res_cw, cw_store,
                                  cw_st_sems, res_slot, origin)
            else:
                compute_and_store(ccw_ref.at[CW_HOPS % 2], res_ccw, ccw_store,
                                  ccw_st_sems, res_slot, origin)
        for slot in (0, 1):
            drain(cw_store, slot)
            drain(ccw_store, slot)

    return pl.pallas_call(
        body,
        out_shape=jax.ShapeDtypeStruct((out_m, n), jnp.float32),
        in_specs=[
            pl.BlockSpec(memory_space=pltpu.VMEM),
            pl.BlockSpec(memory_space=pltpu.VMEM),
        ],
        out_specs=pl.BlockSpec(memory_space=pl.ANY),
        scratch_shapes=[
            pltpu.VMEM((2, m_per, k), jnp.bfloat16),
            pltpu.VMEM((2, m_per, k), jnp.bfloat16),
            pltpu.SemaphoreType.DMA((2,)),
            pltpu.SemaphoreType.DMA((2,)),
            pltpu.SemaphoreType.DMA((2,)),
            pltpu.SemaphoreType.DMA((2,)),
            pltpu.VMEM((2, m_per, n), jnp.float32),
            pltpu.VMEM((2, m_per, n), jnp.float32),
            pltpu.SemaphoreType.DMA((2,)),
            pltpu.SemaphoreType.DMA((2,)),
            pltpu.VMEM((k, n), jnp.bfloat16),
            pltpu.SemaphoreType.REGULAR,
            pltpu.SemaphoreType.REGULAR,
        ],
        compiler_params=pltpu.CompilerParams(collective_id=0),
    )(A, B)


# device time: 188401 ns/iter; 4.2999x vs baseline; 1.0001x over previous
import jax
import jax.numpy as jnp
from jax import lax
from jax.experimental import pallas as pl
from jax.experimental.pallas import tpu as pltpu

N_DEV = 16
CW_HOPS = 8
CCW_HOPS = 7


def kernel(A, B):
    m_per, k = A.shape
    _, n = B.shape
    out_m = N_DEV * m_per

    def body(a_ref, b_ref, out_ref,
             cw_ref, ccw_ref,
             cw_send, cw_recv, ccw_send, ccw_recv,
             res_ref, st0_sems, st1_sems, b_bf,
             cw_credit, ccw_credit):
        my = lax.axis_index("i")
        left = (my - 1) % N_DEV
        right = (my + 1) % N_DEV

        barrier_sem = pltpu.get_barrier_semaphore()
        for nbr in (left, right):
            pl.semaphore_signal(
                barrier_sem, inc=1,
                device_id=(nbr,), device_id_type=pl.DeviceIdType.MESH,
            )
        pl.semaphore_wait(barrier_sem, 2)

        a_bf = a_ref[:, :].astype(jnp.bfloat16)
        b_bf[:, :] = b_ref[:, :].astype(jnp.bfloat16)

        cw_ref[0, :, :] = a_bf
        ccw_ref[0, :, :] = a_bf

        pl.semaphore_signal(
            cw_credit, inc=1,
            device_id=(left,), device_id_type=pl.DeviceIdType.MESH,
        )
        pl.semaphore_signal(
            ccw_credit, inc=1,
            device_id=(right,), device_id_type=pl.DeviceIdType.MESH,
        )

        stores = {}

        def drain(key):
            if key in stores:
                stores[key].wait()
                del stores[key]

        def store_half(res_slot, half, origin):
            sems = st0_sems if half == 0 else st1_sems
            cp = pltpu.make_async_copy(
                res_ref.at[res_slot, pl.ds(half * m_per, m_per), :],
                out_ref.at[pl.ds(origin * m_per, m_per), :],
                sems.at[res_slot],
            )
            cp.start()
            stores[(res_slot, half)] = cp

        for h in range(CW_HOPS):
            ss = h % 2
            rs = (h + 1) % 2
            res_slot = h % 2

            pl.semaphore_wait(cw_credit, 1)
            cw_rdma = pltpu.make_async_remote_copy(
                src_ref=cw_ref.at[ss],
                dst_ref=cw_ref.at[rs],
                send_sem=cw_send.at[ss],
                recv_sem=cw_recv.at[rs],
                device_id=(right,),
                device_id_type=pl.DeviceIdType.MESH,
            )
            cw_rdma.start()
            if h < CCW_HOPS:
                pl.semaphore_wait(ccw_credit, 1)
                ccw_rdma = pltpu.make_async_remote_copy(
                    src_ref=ccw_ref.at[ss],
                    dst_ref=ccw_ref.at[rs],
                    send_sem=ccw_send.at[ss],
                    recv_sem=ccw_recv.at[rs],
                    device_id=(left,),
                    device_id_type=pl.DeviceIdType.MESH,
                )
                ccw_rdma.start()

            drain((res_slot, 0))
            drain((res_slot, 1))
            if h == 0:
                res_ref[0, pl.ds(0, m_per), :] = jnp.dot(
                    cw_ref[0, :, :], b_bf[:, :],
                    preferred_element_type=jnp.float32,
                )
                store_half(0, 0, my)
            else:
                ab = jnp.concatenate(
                    [cw_ref[ss, :, :], ccw_ref[ss, :, :]], axis=0
                )
                res_ref[res_slot, :, :] = jnp.dot(
                    ab, b_bf[:, :], preferred_element_type=jnp.float32
                )
                store_half(res_slot, 0, (my - h) % N_DEV)
                store_half(res_slot, 1, (my + h) % N_DEV)

            cw_rdma.wait()
            if h < CW_HOPS - 1:
                pl.semaphore_signal(
                    cw_credit, inc=1,
                    device_id=(left,), device_id_type=pl.DeviceIdType.MESH,
                )
            if h < CCW_HOPS:
                ccw_rdma.wait()
                if h < CCW_HOPS - 1:
                    pl.semaphore_signal(
                        ccw_credit, inc=1,
                        device_id=(right,), device_id_type=pl.DeviceIdType.MESH,
                    )

        ts = CW_HOPS % 2
        drain((ts, 0))
        res_ref[ts, pl.ds(0, m_per), :] = jnp.dot(
            cw_ref[ts, :, :], b_bf[:, :],
            preferred_element_type=jnp.float32,
        )
        store_half(ts, 0, (my - CW_HOPS) % N_DEV)
        for key in list(stores):
            drain(key)

    return pl.pallas_call(
        body,
        out_shape=jax.ShapeDtypeStruct((out_m, n), jnp.float32),
        in_specs=[
            pl.BlockSpec(memory_space=pltpu.VMEM),
            pl.BlockSpec(memory_space=pltpu.VMEM),
        ],
        out_specs=pl.BlockSpec(memory_space=pl.ANY),
        scratch_shapes=[
            pltpu.VMEM((2, m_per, k), jnp.bfloat16),
            pltpu.VMEM((2, m_per, k), jnp.bfloat16),
            pltpu.SemaphoreType.DMA((2,)),
            pltpu.SemaphoreType.DMA((2,)),
            pltpu.SemaphoreType.DMA((2,)),
            pltpu.SemaphoreType.DMA((2,)),
            pltpu.VMEM((2, 2 * m_per, n), jnp.float32),
            pltpu.SemaphoreType.DMA((2,)),
            pltpu.SemaphoreType.DMA((2,)),
            pltpu.VMEM((k, n), jnp.bfloat16),
            pltpu.SemaphoreType.REGULAR,
            pltpu.SemaphoreType.REGULAR,
        ],
        compiler_params=pltpu.CompilerParams(collective_id=0),
    )(A, B)
